# baseline (speedup 1.0000x reference)
import functools

import numpy as np

import jax
import jax.numpy as jnp
from jax import lax
from jax.experimental import pallas as pl
from jax.experimental.pallas import tpu as pltpu
from jax.experimental.pallas import tpu_sc as plsc

BETA = 0.25

_TM = 512
_TK = 1024

_NC = 2
_NS = 16
_NW = _NC * _NS
_CHUNK = 128

_IOTA_COL = np.arange(_TK, dtype=np.float32)[:, None]


def _prep_body(w_ref, w2_ref, wsq_ref):
    w = w_ref[...]
    w2_ref[...] = w + w
    wsq_ref[...] = jnp.sum(w * w, axis=1, keepdims=True)


def _argmin_body(z_ref, w2_ref, wsq_ref, idx_ref, loss_ref,
                 acc_v0, acc_j0, acc_v1, acc_j1, zsq_s, *, nk, ni, scale):
    i = pl.program_id(0)
    k = pl.program_id(1)
    z = z_ref[...]
    w2 = w2_ref[...]
    tm = z.shape[0]

    @pl.when(k == 0)
    def _init():
        u = z * z
        ones = jnp.ones((1, u.shape[1]), jnp.float32)
        zsq_s[...] = lax.dot_general(ones, u, (((1,), (1,)), ((), ())),
                                     preferred_element_type=jnp.float32)

    dot2 = lax.dot_general(w2, z, (((1,), (1,)), ((), ())),
                           preferred_element_type=jnp.float32)
    zrow = zsq_s[...]
    wcol = wsq_ref[...]

    inf8 = jnp.full((8, tm), jnp.inf, jnp.float32)
    zero8 = jnp.zeros((8, tm), jnp.float32)
    av0 = jnp.where(k == 0, inf8, acc_v0[...])
    aj0 = jnp.where(k == 0, zero8, acc_j0[...])
    av1 = jnp.where(k == 0, inf8, acc_v1[...])
    aj1 = jnp.where(k == 0, zero8, acc_j1[...])
    svec = (lax.broadcasted_iota(jnp.int32, (8, tm), 0).astype(jnp.float32)
            + (k * _TK).astype(jnp.float32))
    for r in range(0, _TK // 8, 2):
        b0 = (zrow - dot2[r * 8:(r + 1) * 8, :]) + wcol[r * 8:(r + 1) * 8, :]
        b1 = (zrow - dot2[(r + 1) * 8:(r + 2) * 8, :]) + wcol[(r + 1) * 8:(r + 2) * 8, :]
        m0 = b0 < av0
        m1 = b1 < av1
        av0 = jnp.where(m0, b0, av0)
        aj0 = jnp.where(m0, svec + jnp.float32(8 * r), aj0)
        av1 = jnp.where(m1, b1, av1)
        aj1 = jnp.where(m1, svec + jnp.float32(8 * (r + 1)), aj1)
    acc_v0[...] = av0
    acc_j0[...] = aj0
    acc_v1[...] = av1
    acc_j1[...] = aj1

    @pl.when(k == nk - 1)
    def _emit():
        take1 = (av1 < av0) | ((av1 == av0) & (aj1 < aj0))
        av = jnp.where(take1, av1, av0)
        aj = jnp.where(take1, aj1, aj0)
        tmin = jnp.min(av, axis=0, keepdims=True)
        jrow = jnp.min(jnp.where(av == tmin, aj, jnp.float32(3.0e38)),
                       axis=0, keepdims=True)
        idx_ref[...] = jrow.astype(jnp.int32)[None]
        part = jnp.sum(tmin, keepdims=True)
        prev = jnp.where(i == 0, jnp.zeros_like(loss_ref[...]), loss_ref[...])
        tot = prev + part
        loss_ref[...] = jnp.where(i == ni - 1, tot * scale, tot)


def _vq_prep(w):
    k_tot, d_dim = w.shape
    nk = k_tot // _TK
    return pl.pallas_call(
        _prep_body,
        grid=(nk,),
        in_specs=[pl.BlockSpec((_TK, d_dim), lambda k: (k, 0))],
        out_specs=[
            pl.BlockSpec((_TK, d_dim), lambda k: (k, 0)),
            pl.BlockSpec((_TK, 1), lambda k: (k, 0)),
        ],
        out_shape=[
            jax.ShapeDtypeStruct((k_tot, d_dim), jnp.float32),
            jax.ShapeDtypeStruct((k_tot, 1), jnp.float32),
        ],
    )(w)


def _vq_argmin(zf, w2, wsq):
    m, d_dim = zf.shape
    k_tot = w2.shape[0]
    ni = m // _TM
    nk = k_tot // _TK
    scale = (1.0 + BETA) / (m * d_dim)
    idx3, loss = pl.pallas_call(
        functools.partial(_argmin_body, nk=nk, ni=ni, scale=scale),
        grid=(ni, nk),
        in_specs=[
            pl.BlockSpec((_TM, d_dim), lambda i, k: (i, 0)),
            pl.BlockSpec((_TK, d_dim), lambda i, k: (k, 0)),
            pl.BlockSpec((_TK, 1), lambda i, k: (k, 0)),
        ],
        out_specs=[
            pl.BlockSpec((1, 1, _TM), lambda i, k: (i, 0, 0)),
            pl.BlockSpec((1, 1), lambda i, k: (0, 0)),
        ],
        out_shape=[
            jax.ShapeDtypeStruct((ni, 1, _TM), jnp.int32),
            jax.ShapeDtypeStruct((1, 1), jnp.float32),
        ],
        scratch_shapes=[
            pltpu.VMEM((8, _TM), jnp.float32),
            pltpu.VMEM((8, _TM), jnp.float32),
            pltpu.VMEM((8, _TM), jnp.float32),
            pltpu.VMEM((8, _TM), jnp.float32),
            pltpu.VMEM((1, _TM), jnp.float32),
        ],
        compiler_params=pltpu.CompilerParams(
            dimension_semantics=("arbitrary", "arbitrary")),
    )(zf, w2, wsq)
    return idx3.reshape(m), loss[0, 0]


def _gather_body(w_hbm, idx_hbm, out_hbm, idx_v, rows_v, sem):
    wid = lax.axis_index("s") * _NC + lax.axis_index("c")
    b_per_w = idx_hbm.shape[0] // _NW
    base = wid * b_per_w
    nch = b_per_w // _CHUNK

    def chunk(c, carry):
        off = base + c * _CHUNK
        pltpu.sync_copy(idx_hbm.at[pl.ds(off, _CHUNK)], idx_v)
        pltpu.async_copy(w_hbm.at[idx_v], rows_v, sem).wait()
        pltpu.sync_copy(rows_v, out_hbm.at[pl.ds(off, _CHUNK)])
        return carry

    lax.fori_loop(0, nch, chunk, 0)


def _vq_gather(w, idx):
    m = idx.shape[0]
    d_dim = w.shape[1]
    mesh = plsc.VectorSubcoreMesh(core_axis_name="c", subcore_axis_name="s")
    fn = functools.partial(
        pl.kernel,
        mesh=mesh,
        out_type=jax.ShapeDtypeStruct((m, d_dim), jnp.float32),
        scratch_types=[
            pltpu.VMEM((_CHUNK,), jnp.int32),
            pltpu.VMEM((_CHUNK, d_dim), jnp.float32),
            pltpu.SemaphoreType.DMA,
        ],
    )(_gather_body)
    return fn(w, idx)


def kernel(z, W):
    zf = z.reshape(-1, z.shape[-1])
    w2, wsq = _vq_prep(W)
    idx, loss = _vq_argmin(zf, w2, wsq)
    z_q = _vq_gather(W, idx)
    z_q_st = z_q.reshape(z.shape)
    min_encoding_indices = idx.reshape(z.shape[:-1] + (1,))
    return (z_q_st, loss, min_encoding_indices)


# speedup vs baseline: 1.2550x; 1.2550x over previous
import functools

import numpy as np

import jax
import jax.numpy as jnp
from jax import lax
from jax.experimental import pallas as pl
from jax.experimental.pallas import tpu as pltpu
from jax.experimental.pallas import tpu_sc as plsc

BETA = 0.25

_TM = 512
_TK = 2048
_TKP = 1024

_NC = 2
_NS = 16
_NW = _NC * _NS
_CHUNK = 128

_IOTA_COL = np.arange(_TK, dtype=np.float32)[:, None]


def _prep_body(w_ref, w2_ref, wsq_ref):
    w = w_ref[...]
    w2_ref[...] = w + w
    wsq_ref[...] = jnp.sum(w * w, axis=1, keepdims=True)


def _prepz_body(z_ref, zsq_ref):
    u = z_ref[...]
    u = u * u
    ones = jnp.ones((1, u.shape[1]), jnp.float32)
    zsq_ref[...] = lax.dot_general(ones, u, (((1,), (1,)), ((), ())),
                                   preferred_element_type=jnp.float32)[None]


def _argmin_body(z_ref, w2_ref, wsq_ref, zsq_ref, idx_ref, loss_ref,
                 acc_v0, acc_j0, acc_v1, acc_j1, *, nk, ni, scale):
    i = pl.program_id(0)
    k = pl.program_id(1)
    z = z_ref[...]
    w2 = w2_ref[...]
    tm = z.shape[0]

    dot2 = lax.dot_general(w2, z, (((1,), (1,)), ((), ())),
                           preferred_element_type=jnp.float32)
    zrow = zsq_ref[0]
    wcol = wsq_ref[...]

    inf8 = jnp.full((8, tm), jnp.inf, jnp.float32)
    zero8 = jnp.zeros((8, tm), jnp.float32)
    av0 = jnp.where(k == 0, inf8, acc_v0[...])
    aj0 = jnp.where(k == 0, zero8, acc_j0[...])
    av1 = jnp.where(k == 0, inf8, acc_v1[...])
    aj1 = jnp.where(k == 0, zero8, acc_j1[...])
    rbase = (k * (_TK // 8)).astype(jnp.float32)
    for r in range(0, _TK // 8, 2):
        b0 = (zrow - dot2[r * 8:(r + 1) * 8, :]) + wcol[r * 8:(r + 1) * 8, :]
        b1 = (zrow - dot2[(r + 1) * 8:(r + 2) * 8, :]) + wcol[(r + 1) * 8:(r + 2) * 8, :]
        m0 = b0 < av0
        m1 = b1 < av1
        av0 = jnp.minimum(b0, av0)
        aj0 = jnp.where(m0, rbase + jnp.float32(r), aj0)
        av1 = jnp.minimum(b1, av1)
        aj1 = jnp.where(m1, rbase + jnp.float32(r + 1), aj1)
    acc_v0[...] = av0
    acc_j0[...] = aj0
    acc_v1[...] = av1
    acc_j1[...] = aj1

    @pl.when(k == nk - 1)
    def _emit():
        svec = lax.broadcasted_iota(jnp.int32, (8, tm), 0).astype(jnp.float32)
        j0 = aj0 * jnp.float32(8.0) + svec
        j1 = aj1 * jnp.float32(8.0) + svec
        take1 = (av1 < av0) | ((av1 == av0) & (j1 < j0))
        av = jnp.where(take1, av1, av0)
        aj = jnp.where(take1, j1, j0)
        tmin = jnp.min(av, axis=0, keepdims=True)
        jrow = jnp.min(jnp.where(av == tmin, aj, jnp.float32(3.0e38)),
                       axis=0, keepdims=True)
        idx_ref[...] = jrow.astype(jnp.int32)[None]
        part = jnp.sum(tmin, keepdims=True)
        prev = jnp.where(i == 0, jnp.zeros_like(loss_ref[...]), loss_ref[...])
        tot = prev + part
        loss_ref[...] = jnp.where(i == ni - 1, tot * scale, tot)


def _vq_prep(w):
    k_tot, d_dim = w.shape
    nk = k_tot // _TKP
    return pl.pallas_call(
        _prep_body,
        grid=(nk,),
        in_specs=[pl.BlockSpec((_TKP, d_dim), lambda k: (k, 0))],
        out_specs=[
            pl.BlockSpec((_TKP, d_dim), lambda k: (k, 0)),
            pl.BlockSpec((_TKP, 1), lambda k: (k, 0)),
        ],
        out_shape=[
            jax.ShapeDtypeStruct((k_tot, d_dim), jnp.float32),
            jax.ShapeDtypeStruct((k_tot, 1), jnp.float32),
        ],
    )(w)


def _vq_prep_z(zf):
    m, d_dim = zf.shape
    ni = m // _TM
    return pl.pallas_call(
        _prepz_body,
        grid=(ni,),
        in_specs=[pl.BlockSpec((_TM, d_dim), lambda i: (i, 0))],
        out_specs=pl.BlockSpec((1, 1, _TM), lambda i: (i, 0, 0)),
        out_shape=jax.ShapeDtypeStruct((ni, 1, _TM), jnp.float32),
    )(zf)


def _vq_argmin(zf, w2, wsq, zsq):
    m, d_dim = zf.shape
    k_tot = w2.shape[0]
    ni = m // _TM
    nk = k_tot // _TK
    scale = (1.0 + BETA) / (m * d_dim)
    idx3, loss = pl.pallas_call(
        functools.partial(_argmin_body, nk=nk, ni=ni, scale=scale),
        grid=(ni, nk),
        in_specs=[
            pl.BlockSpec((_TM, d_dim), lambda i, k: (i, 0)),
            pl.BlockSpec((_TK, d_dim), lambda i, k: (k, 0)),
            pl.BlockSpec((_TK, 1), lambda i, k: (k, 0)),
            pl.BlockSpec((1, 1, _TM), lambda i, k: (i, 0, 0)),
        ],
        out_specs=[
            pl.BlockSpec((1, 1, _TM), lambda i, k: (i, 0, 0)),
            pl.BlockSpec((1, 1), lambda i, k: (0, 0)),
        ],
        out_shape=[
            jax.ShapeDtypeStruct((ni, 1, _TM), jnp.int32),
            jax.ShapeDtypeStruct((1, 1), jnp.float32),
        ],
        scratch_shapes=[
            pltpu.VMEM((8, _TM), jnp.float32),
            pltpu.VMEM((8, _TM), jnp.float32),
            pltpu.VMEM((8, _TM), jnp.float32),
            pltpu.VMEM((8, _TM), jnp.float32),
        ],
        compiler_params=pltpu.CompilerParams(
            dimension_semantics=("arbitrary", "arbitrary")),
    )(zf, w2, wsq, zsq)
    return idx3.reshape(m), loss[0, 0]


def _gather_body(w_hbm, idx_hbm, out_hbm, idx_v, rows_v, sem):
    wid = lax.axis_index("s") * _NC + lax.axis_index("c")
    b_per_w = idx_hbm.shape[0] // _NW
    base = wid * b_per_w
    nch = b_per_w // _CHUNK

    def chunk(c, carry):
        off = base + c * _CHUNK
        pltpu.sync_copy(idx_hbm.at[pl.ds(off, _CHUNK)], idx_v)
        pltpu.async_copy(w_hbm.at[idx_v], rows_v, sem).wait()
        pltpu.sync_copy(rows_v, out_hbm.at[pl.ds(off, _CHUNK)])
        return carry

    lax.fori_loop(0, nch, chunk, 0)


def _vq_gather(w, idx):
    m = idx.shape[0]
    d_dim = w.shape[1]
    mesh = plsc.VectorSubcoreMesh(core_axis_name="c", subcore_axis_name="s")
    fn = functools.partial(
        pl.kernel,
        mesh=mesh,
        out_type=jax.ShapeDtypeStruct((m, d_dim), jnp.float32),
        scratch_types=[
            pltpu.VMEM((_CHUNK,), jnp.int32),
            pltpu.VMEM((_CHUNK, d_dim), jnp.float32),
            pltpu.SemaphoreType.DMA,
        ],
    )(_gather_body)
    return fn(w, idx)


def kernel(z, W):
    zf = z.reshape(-1, z.shape[-1])
    w2, wsq = _vq_prep(W)
    zsq = _vq_prep_z(zf)
    idx, loss = _vq_argmin(zf, w2, wsq, zsq)
    z_q = _vq_gather(W, idx)
    z_q_st = z_q.reshape(z.shape)
    min_encoding_indices = idx.reshape(z.shape[:-1] + (1,))
    return (z_q_st, loss, min_encoding_indices)


# speedup vs baseline: 1.5516x; 1.2364x over previous
import functools

import numpy as np

import jax
import jax.numpy as jnp
from jax import lax
from jax.experimental import pallas as pl
from jax.experimental.pallas import tpu as pltpu
from jax.experimental.pallas import tpu_sc as plsc

BETA = 0.25

_TM = 512
_TK = 2048
_TKP = 1024

_NC = 2
_NS = 16
_NW = _NC * _NS
_CHUNK = 128

_IOTA_COL = np.arange(_TK, dtype=np.float32)[:, None]


def _prep_body(w_ref, w2_ref, wsq_ref):
    w = w_ref[...]
    w2_ref[...] = w + w
    wsq_ref[...] = jnp.sum(w * w, axis=1, keepdims=True)


def _prepz_body(z_ref, zsq_ref):
    u = z_ref[...]
    u = u * u
    ones = jnp.ones((1, u.shape[1]), jnp.float32)
    zsq_ref[...] = lax.dot_general(ones, u, (((1,), (1,)), ((), ())),
                                   preferred_element_type=jnp.float32)[None]


def _argmin_body(z_ref, w2_ref, wsq_ref, zsq_ref, idx_ref, loss_ref,
                 acc_v0, acc_j0, acc_v1, acc_j1, *, nk, ni, scale):
    k = pl.program_id(0)
    i = pl.program_id(1)
    z = z_ref[...]
    w2 = w2_ref[...]
    tm = z.shape[0]

    dot2 = lax.dot_general(w2, z, (((1,), (1,)), ((), ())),
                           preferred_element_type=jnp.float32)
    zrow = zsq_ref[0]
    wcol = wsq_ref[...]

    inf8 = jnp.full((8, tm), jnp.inf, jnp.float32)
    zero8 = jnp.zeros((8, tm), jnp.float32)
    av0 = jnp.where(k == 0, inf8, acc_v0[i])
    aj0 = jnp.where(k == 0, zero8, acc_j0[i])
    av1 = jnp.where(k == 0, inf8, acc_v1[i])
    aj1 = jnp.where(k == 0, zero8, acc_j1[i])
    rbase = (k * (_TK // 8)).astype(jnp.float32)
    for r in range(0, _TK // 8, 2):
        b0 = (zrow - dot2[r * 8:(r + 1) * 8, :]) + wcol[r * 8:(r + 1) * 8, :]
        b1 = (zrow - dot2[(r + 1) * 8:(r + 2) * 8, :]) + wcol[(r + 1) * 8:(r + 2) * 8, :]
        m0 = b0 < av0
        m1 = b1 < av1
        av0 = jnp.minimum(b0, av0)
        aj0 = jnp.where(m0, rbase + jnp.float32(r), aj0)
        av1 = jnp.minimum(b1, av1)
        aj1 = jnp.where(m1, rbase + jnp.float32(r + 1), aj1)
    acc_v0[i] = av0
    acc_j0[i] = aj0
    acc_v1[i] = av1
    acc_j1[i] = aj1

    @pl.when(k == nk - 1)
    def _emit():
        svec = lax.broadcasted_iota(jnp.int32, (8, tm), 0).astype(jnp.float32)
        j0 = aj0 * jnp.float32(8.0) + svec
        j1 = aj1 * jnp.float32(8.0) + svec
        take1 = (av1 < av0) | ((av1 == av0) & (j1 < j0))
        av = jnp.where(take1, av1, av0)
        aj = jnp.where(take1, j1, j0)
        tmin = jnp.min(av, axis=0, keepdims=True)
        jrow = jnp.min(jnp.where(av == tmin, aj, jnp.float32(3.0e38)),
                       axis=0, keepdims=True)
        idx_ref[...] = jrow.astype(jnp.int32)[None]
        part = jnp.sum(tmin, keepdims=True)
        prev = jnp.where(i == 0, jnp.zeros_like(loss_ref[...]), loss_ref[...])
        tot = prev + part
        loss_ref[...] = jnp.where(i == ni - 1, tot * scale, tot)


def _vq_prep(w):
    k_tot, d_dim = w.shape
    nk = k_tot // _TKP
    return pl.pallas_call(
        _prep_body,
        grid=(nk,),
        in_specs=[pl.BlockSpec((_TKP, d_dim), lambda k: (k, 0))],
        out_specs=[
            pl.BlockSpec((_TKP, d_dim), lambda k: (k, 0)),
            pl.BlockSpec((_TKP, 1), lambda k: (k, 0)),
        ],
        out_shape=[
            jax.ShapeDtypeStruct((k_tot, d_dim), jnp.float32),
            jax.ShapeDtypeStruct((k_tot, 1), jnp.float32),
        ],
    )(w)


def _vq_prep_z(zf):
    m, d_dim = zf.shape
    ni = m // _TM
    return pl.pallas_call(
        _prepz_body,
        grid=(ni,),
        in_specs=[pl.BlockSpec((_TM, d_dim), lambda i: (i, 0))],
        out_specs=pl.BlockSpec((1, 1, _TM), lambda i: (i, 0, 0)),
        out_shape=jax.ShapeDtypeStruct((ni, 1, _TM), jnp.float32),
    )(zf)


def _vq_argmin(zf, w2, wsq, zsq):
    m, d_dim = zf.shape
    k_tot = w2.shape[0]
    ni = m // _TM
    nk = k_tot // _TK
    scale = (1.0 + BETA) / (m * d_dim)
    idx3, loss = pl.pallas_call(
        functools.partial(_argmin_body, nk=nk, ni=ni, scale=scale),
        grid=(nk, ni),
        in_specs=[
            pl.BlockSpec((_TM, d_dim), lambda k, i: (i, 0)),
            pl.BlockSpec((_TK, d_dim), lambda k, i: (k, 0)),
            pl.BlockSpec((_TK, 1), lambda k, i: (k, 0)),
            pl.BlockSpec((1, 1, _TM), lambda k, i: (i, 0, 0)),
        ],
        out_specs=[
            pl.BlockSpec((1, 1, _TM), lambda k, i: (i, 0, 0)),
            pl.BlockSpec((1, 1), lambda k, i: (0, 0)),
        ],
        out_shape=[
            jax.ShapeDtypeStruct((ni, 1, _TM), jnp.int32),
            jax.ShapeDtypeStruct((1, 1), jnp.float32),
        ],
        scratch_shapes=[
            pltpu.VMEM((ni, 8, _TM), jnp.float32),
            pltpu.VMEM((ni, 8, _TM), jnp.float32),
            pltpu.VMEM((ni, 8, _TM), jnp.float32),
            pltpu.VMEM((ni, 8, _TM), jnp.float32),
        ],
        compiler_params=pltpu.CompilerParams(
            dimension_semantics=("arbitrary", "arbitrary")),
    )(zf, w2, wsq, zsq)
    return idx3.reshape(m), loss[0, 0]


def _gather_body(w_hbm, idx_hbm, out_hbm, idx_v, rows_v, sem):
    wid = lax.axis_index("s") * _NC + lax.axis_index("c")
    b_per_w = idx_hbm.shape[0] // _NW
    base = wid * b_per_w
    nch = b_per_w // _CHUNK

    def chunk(c, carry):
        off = base + c * _CHUNK
        pltpu.sync_copy(idx_hbm.at[pl.ds(off, _CHUNK)], idx_v)
        pltpu.async_copy(w_hbm.at[idx_v], rows_v, sem).wait()
        pltpu.sync_copy(rows_v, out_hbm.at[pl.ds(off, _CHUNK)])
        return carry

    lax.fori_loop(0, nch, chunk, 0)


def _vq_gather(w, idx):
    m = idx.shape[0]
    d_dim = w.shape[1]
    mesh = plsc.VectorSubcoreMesh(core_axis_name="c", subcore_axis_name="s")
    fn = functools.partial(
        pl.kernel,
        mesh=mesh,
        out_type=jax.ShapeDtypeStruct((m, d_dim), jnp.float32),
        scratch_types=[
            pltpu.VMEM((_CHUNK,), jnp.int32),
            pltpu.VMEM((_CHUNK, d_dim), jnp.float32),
            pltpu.SemaphoreType.DMA,
        ],
    )(_gather_body)
    return fn(w, idx)


def kernel(z, W):
    zf = z.reshape(-1, z.shape[-1])
    w2, wsq = _vq_prep(W)
    zsq = _vq_prep_z(zf)
    idx, loss = _vq_argmin(zf, w2, wsq, zsq)
    z_q = _vq_gather(W, idx)
    z_q_st = z_q.reshape(z.shape)
    min_encoding_indices = idx.reshape(z.shape[:-1] + (1,))
    return (z_q_st, loss, min_encoding_indices)


# speedup vs baseline: 1.9135x; 1.2332x over previous
import functools

import numpy as np

import jax
import jax.numpy as jnp
from jax import lax
from jax.experimental import pallas as pl
from jax.experimental.pallas import tpu as pltpu
from jax.experimental.pallas import tpu_sc as plsc

BETA = 0.25

_TM = 512
_TK = 8192
_TKP = 1024

_NC = 2
_NS = 16
_NW = _NC * _NS
_CHUNK = 128

_IOTA_COL = np.arange(_TK, dtype=np.float32)[:, None]


def _prep_body(w_ref, w2_ref, wsq_ref):
    w = w_ref[...]
    w2_ref[...] = w + w
    wsq_ref[...] = jnp.sum(w * w, axis=1, keepdims=True)


def _prepz_body(z_ref, zsq_ref):
    u = z_ref[...]
    u = u * u
    ones = jnp.ones((1, u.shape[1]), jnp.float32)
    zsq_ref[...] = lax.dot_general(ones, u, (((1,), (1,)), ((), ())),
                                   preferred_element_type=jnp.float32)[None]


def _argmin_body(z_ref, w2_ref, wsq_ref, zsq_ref, idx_ref, loss_ref,
                 acc_v0, acc_j0, acc_v1, acc_j1, *, nk, ni, scale):
    k = pl.program_id(0)
    i = pl.program_id(1)
    z = z_ref[...]
    w2 = w2_ref[...]
    tm = z.shape[0]

    dot2 = lax.dot_general(w2, z, (((1,), (1,)), ((), ())),
                           preferred_element_type=jnp.float32)
    zrow = zsq_ref[0]
    wcol = wsq_ref[...]

    inf8 = jnp.full((8, tm), jnp.inf, jnp.float32)
    zero8 = jnp.zeros((8, tm), jnp.float32)
    av0 = jnp.where(k == 0, inf8, acc_v0[i])
    aj0 = jnp.where(k == 0, zero8, acc_j0[i])
    av1 = jnp.where(k == 0, inf8, acc_v1[i])
    aj1 = jnp.where(k == 0, zero8, acc_j1[i])
    rbase = (k * (_TK // 8)).astype(jnp.float32)
    for r in range(0, _TK // 8, 2):
        b0 = (zrow - dot2[r * 8:(r + 1) * 8, :]) + wcol[r * 8:(r + 1) * 8, :]
        b1 = (zrow - dot2[(r + 1) * 8:(r + 2) * 8, :]) + wcol[(r + 1) * 8:(r + 2) * 8, :]
        m0 = b0 < av0
        m1 = b1 < av1
        av0 = jnp.minimum(b0, av0)
        aj0 = jnp.where(m0, rbase + jnp.float32(r), aj0)
        av1 = jnp.minimum(b1, av1)
        aj1 = jnp.where(m1, rbase + jnp.float32(r + 1), aj1)
    acc_v0[i] = av0
    acc_j0[i] = aj0
    acc_v1[i] = av1
    acc_j1[i] = aj1

    @pl.when(k == nk - 1)
    def _emit():
        svec = lax.broadcasted_iota(jnp.int32, (8, tm), 0).astype(jnp.float32)
        j0 = aj0 * jnp.float32(8.0) + svec
        j1 = aj1 * jnp.float32(8.0) + svec
        take1 = (av1 < av0) | ((av1 == av0) & (j1 < j0))
        av = jnp.where(take1, av1, av0)
        aj = jnp.where(take1, j1, j0)
        tmin = jnp.min(av, axis=0, keepdims=True)
        jrow = jnp.min(jnp.where(av == tmin, aj, jnp.float32(3.0e38)),
                       axis=0, keepdims=True)
        idx_ref[...] = jrow.astype(jnp.int32)[None]
        part = jnp.sum(tmin, keepdims=True)
        prev = jnp.where(i == 0, jnp.zeros_like(loss_ref[...]), loss_ref[...])
        tot = prev + part
        loss_ref[...] = jnp.where(i == ni - 1, tot * scale, tot)


def _vq_prep(w):
    k_tot, d_dim = w.shape
    nk = k_tot // _TKP
    return pl.pallas_call(
        _prep_body,
        grid=(nk,),
        in_specs=[pl.BlockSpec((_TKP, d_dim), lambda k: (k, 0))],
        out_specs=[
            pl.BlockSpec((_TKP, d_dim), lambda k: (k, 0)),
            pl.BlockSpec((_TKP, 1), lambda k: (k, 0)),
        ],
        out_shape=[
            jax.ShapeDtypeStruct((k_tot, d_dim), jnp.float32),
            jax.ShapeDtypeStruct((k_tot, 1), jnp.float32),
        ],
    )(w)


def _vq_prep_z(zf):
    m, d_dim = zf.shape
    ni = m // _TM
    return pl.pallas_call(
        _prepz_body,
        grid=(ni,),
        in_specs=[pl.BlockSpec((_TM, d_dim), lambda i: (i, 0))],
        out_specs=pl.BlockSpec((1, 1, _TM), lambda i: (i, 0, 0)),
        out_shape=jax.ShapeDtypeStruct((ni, 1, _TM), jnp.float32),
    )(zf)


def _vq_argmin(zf, w2, wsq, zsq):
    m, d_dim = zf.shape
    k_tot = w2.shape[0]
    ni = m // _TM
    nk = k_tot // _TK
    scale = (1.0 + BETA) / (m * d_dim)
    idx3, loss = pl.pallas_call(
        functools.partial(_argmin_body, nk=nk, ni=ni, scale=scale),
        grid=(nk, ni),
        in_specs=[
            pl.BlockSpec((_TM, d_dim), lambda k, i: (i, 0)),
            pl.BlockSpec((_TK, d_dim), lambda k, i: (k, 0)),
            pl.BlockSpec((_TK, 1), lambda k, i: (k, 0)),
            pl.BlockSpec((1, 1, _TM), lambda k, i: (i, 0, 0)),
        ],
        out_specs=[
            pl.BlockSpec((1, 1, _TM), lambda k, i: (i, 0, 0)),
            pl.BlockSpec((1, 1), lambda k, i: (0, 0)),
        ],
        out_shape=[
            jax.ShapeDtypeStruct((ni, 1, _TM), jnp.int32),
            jax.ShapeDtypeStruct((1, 1), jnp.float32),
        ],
        scratch_shapes=[
            pltpu.VMEM((ni, 8, _TM), jnp.float32),
            pltpu.VMEM((ni, 8, _TM), jnp.float32),
            pltpu.VMEM((ni, 8, _TM), jnp.float32),
            pltpu.VMEM((ni, 8, _TM), jnp.float32),
        ],
        compiler_params=pltpu.CompilerParams(
            dimension_semantics=("arbitrary", "arbitrary")),
    )(zf, w2, wsq, zsq)
    return idx3.reshape(m), loss[0, 0]


def _gather_body(w_hbm, idx_hbm, out_hbm, idx_v, rows_v, sem):
    wid = lax.axis_index("s") * _NC + lax.axis_index("c")
    b_per_w = idx_hbm.shape[0] // _NW
    base = wid * b_per_w
    nch = b_per_w // _CHUNK

    def chunk(c, carry):
        off = base + c * _CHUNK
        pltpu.sync_copy(idx_hbm.at[pl.ds(off, _CHUNK)], idx_v)
        pltpu.async_copy(w_hbm.at[idx_v], rows_v, sem).wait()
        pltpu.sync_copy(rows_v, out_hbm.at[pl.ds(off, _CHUNK)])
        return carry

    lax.fori_loop(0, nch, chunk, 0)


def _vq_gather(w, idx):
    m = idx.shape[0]
    d_dim = w.shape[1]
    mesh = plsc.VectorSubcoreMesh(core_axis_name="c", subcore_axis_name="s")
    fn = functools.partial(
        pl.kernel,
        mesh=mesh,
        out_type=jax.ShapeDtypeStruct((m, d_dim), jnp.float32),
        scratch_types=[
            pltpu.VMEM((_CHUNK,), jnp.int32),
            pltpu.VMEM((_CHUNK, d_dim), jnp.float32),
            pltpu.SemaphoreType.DMA,
        ],
    )(_gather_body)
    return fn(w, idx)


def kernel(z, W):
    zf = z.reshape(-1, z.shape[-1])
    w2, wsq = _vq_prep(W)
    zsq = _vq_prep_z(zf)
    idx, loss = _vq_argmin(zf, w2, wsq, zsq)
    z_q = _vq_gather(W, idx)
    z_q_st = z_q.reshape(z.shape)
    min_encoding_indices = idx.reshape(z.shape[:-1] + (1,))
    return (z_q_st, loss, min_encoding_indices)


# speedup vs baseline: 2.1586x; 1.1281x over previous
import functools

import numpy as np

import jax
import jax.numpy as jnp
from jax import lax
from jax.experimental import pallas as pl
from jax.experimental.pallas import tpu as pltpu
from jax.experimental.pallas import tpu_sc as plsc

BETA = 0.25

_TM = 512
_TK = 8192
_TKP = 1024

_NC = 2
_NS = 16
_NW = _NC * _NS
_CHUNK = 128

_IOTA_COL = np.arange(_TK, dtype=np.float32)[:, None]


def _prep_body(w_ref, w2_ref, wsq_ref):
    w = w_ref[...]
    w2_ref[...] = w + w
    wsq_ref[...] = jnp.sum(w * w, axis=1, keepdims=True)


def _argmin_body(z_ref, w2_ref, wsq_ref, idx_ref, loss_ref, *, ni, scale):
    i = pl.program_id(0)
    z = z_ref[...]
    w2 = w2_ref[...]
    tm = z.shape[0]

    u = z * z
    ones = jnp.ones((1, u.shape[1]), jnp.float32)
    zrow = lax.dot_general(ones, u, (((1,), (1,)), ((), ())),
                           preferred_element_type=jnp.float32)

    dot2 = lax.dot_general(w2, z, (((1,), (1,)), ((), ())),
                           preferred_element_type=jnp.float32)
    wcol = wsq_ref[...]

    av0 = jnp.full((8, tm), jnp.inf, jnp.float32)
    aj0 = jnp.zeros((8, tm), jnp.float32)
    av1 = jnp.full((8, tm), jnp.inf, jnp.float32)
    aj1 = jnp.zeros((8, tm), jnp.float32)
    for r in range(0, _TK // 8, 2):
        b0 = (zrow - dot2[r * 8:(r + 1) * 8, :]) + wcol[r * 8:(r + 1) * 8, :]
        b1 = (zrow - dot2[(r + 1) * 8:(r + 2) * 8, :]) + wcol[(r + 1) * 8:(r + 2) * 8, :]
        m0 = b0 < av0
        m1 = b1 < av1
        av0 = jnp.minimum(b0, av0)
        aj0 = jnp.where(m0, jnp.float32(r), aj0)
        av1 = jnp.minimum(b1, av1)
        aj1 = jnp.where(m1, jnp.float32(r + 1), aj1)

    svec = lax.broadcasted_iota(jnp.int32, (8, tm), 0).astype(jnp.float32)
    j0 = aj0 * jnp.float32(8.0) + svec
    j1 = aj1 * jnp.float32(8.0) + svec
    take1 = (av1 < av0) | ((av1 == av0) & (j1 < j0))
    av = jnp.where(take1, av1, av0)
    aj = jnp.where(take1, j1, j0)
    tmin = jnp.min(av, axis=0, keepdims=True)
    jrow = jnp.min(jnp.where(av == tmin, aj, jnp.float32(3.0e38)),
                   axis=0, keepdims=True)
    idx_ref[...] = jrow.astype(jnp.int32)[None]
    part = jnp.sum(tmin, keepdims=True)
    prev = jnp.where(i == 0, jnp.zeros_like(loss_ref[...]), loss_ref[...])
    tot = prev + part
    loss_ref[...] = jnp.where(i == ni - 1, tot * scale, tot)


def _vq_prep(w):
    k_tot, d_dim = w.shape
    nk = k_tot // _TKP
    return pl.pallas_call(
        _prep_body,
        grid=(nk,),
        in_specs=[pl.BlockSpec((_TKP, d_dim), lambda k: (k, 0))],
        out_specs=[
            pl.BlockSpec((_TKP, d_dim), lambda k: (k, 0)),
            pl.BlockSpec((_TKP, 1), lambda k: (k, 0)),
        ],
        out_shape=[
            jax.ShapeDtypeStruct((k_tot, d_dim), jnp.float32),
            jax.ShapeDtypeStruct((k_tot, 1), jnp.float32),
        ],
    )(w)


def _vq_argmin(zf, w2, wsq):
    m, d_dim = zf.shape
    k_tot = w2.shape[0]
    ni = m // _TM
    scale = (1.0 + BETA) / (m * d_dim)
    idx3, loss = pl.pallas_call(
        functools.partial(_argmin_body, ni=ni, scale=scale),
        grid=(ni,),
        in_specs=[
            pl.BlockSpec((_TM, d_dim), lambda i: (i, 0)),
            pl.BlockSpec((_TK, d_dim), lambda i: (0, 0)),
            pl.BlockSpec((_TK, 1), lambda i: (0, 0)),
        ],
        out_specs=[
            pl.BlockSpec((1, 1, _TM), lambda i: (i, 0, 0)),
            pl.BlockSpec((1, 1), lambda i: (0, 0)),
        ],
        out_shape=[
            jax.ShapeDtypeStruct((ni, 1, _TM), jnp.int32),
            jax.ShapeDtypeStruct((1, 1), jnp.float32),
        ],
        compiler_params=pltpu.CompilerParams(
            dimension_semantics=("arbitrary",)),
    )(zf, w2, wsq)
    return idx3.reshape(m), loss[0, 0]


def _gather_body(w_hbm, idx_hbm, out_hbm, idx_v, rows_v, sem):
    wid = lax.axis_index("s") * _NC + lax.axis_index("c")
    b_per_w = idx_hbm.shape[0] // _NW
    base = wid * b_per_w
    nch = b_per_w // _CHUNK

    def chunk(c, carry):
        off = base + c * _CHUNK
        pltpu.sync_copy(idx_hbm.at[pl.ds(off, _CHUNK)], idx_v)
        pltpu.async_copy(w_hbm.at[idx_v], rows_v, sem).wait()
        pltpu.sync_copy(rows_v, out_hbm.at[pl.ds(off, _CHUNK)])
        return carry

    lax.fori_loop(0, nch, chunk, 0)


def _vq_gather(w, idx):
    m = idx.shape[0]
    d_dim = w.shape[1]
    mesh = plsc.VectorSubcoreMesh(core_axis_name="c", subcore_axis_name="s")
    fn = functools.partial(
        pl.kernel,
        mesh=mesh,
        out_type=jax.ShapeDtypeStruct((m, d_dim), jnp.float32),
        scratch_types=[
            pltpu.VMEM((_CHUNK,), jnp.int32),
            pltpu.VMEM((_CHUNK, d_dim), jnp.float32),
            pltpu.SemaphoreType.DMA,
        ],
    )(_gather_body)
    return fn(w, idx)


def kernel(z, W):
    zf = z.reshape(-1, z.shape[-1])
    w2, wsq = _vq_prep(W)
    idx, loss = _vq_argmin(zf, w2, wsq)
    z_q = _vq_gather(W, idx)
    z_q_st = z_q.reshape(z.shape)
    min_encoding_indices = idx.reshape(z.shape[:-1] + (1,))
    return (z_q_st, loss, min_encoding_indices)


# speedup vs baseline: 2.1909x; 1.0149x over previous
import functools

import numpy as np

import jax
import jax.numpy as jnp
from jax import lax
from jax.experimental import pallas as pl
from jax.experimental.pallas import tpu as pltpu
from jax.experimental.pallas import tpu_sc as plsc

BETA = 0.25

_TM = 512
_TK = 8192
_TKP = 1024

_NC = 2
_NS = 16
_NW = _NC * _NS
_CHUNK = 128

_IOTA_COL = np.arange(_TK, dtype=np.float32)[:, None]


def _prep_body(w_ref, w2_ref, wsq_ref):
    w = w_ref[...]
    w2_ref[...] = w + w
    wsq_ref[...] = jnp.sum(w * w, axis=1, keepdims=True)


def _argmin_body(z_ref, w2_ref, wsq_ref, idx_ref, loss_ref, *, ni, scale):
    i = pl.program_id(0)
    z = z_ref[...]
    w2 = w2_ref[...]
    tm = z.shape[0]

    u = z * z
    ones = jnp.ones((1, u.shape[1]), jnp.float32)
    zrow = lax.dot_general(ones, u, (((1,), (1,)), ((), ())),
                           preferred_element_type=jnp.float32)

    dot2 = lax.dot_general(w2, z, (((1,), (1,)), ((), ())),
                           preferred_element_type=jnp.float32)
    wcol = wsq_ref[...]

    av0 = jnp.full((8, tm), jnp.inf, jnp.float32)
    aj0 = jnp.zeros((8, tm), jnp.float32)
    av1 = jnp.full((8, tm), jnp.inf, jnp.float32)
    aj1 = jnp.zeros((8, tm), jnp.float32)
    for r in range(0, _TK // 8, 2):
        b0 = (zrow - dot2[r * 8:(r + 1) * 8, :]) + wcol[r * 8:(r + 1) * 8, :]
        b1 = (zrow - dot2[(r + 1) * 8:(r + 2) * 8, :]) + wcol[(r + 1) * 8:(r + 2) * 8, :]
        m0 = b0 < av0
        m1 = b1 < av1
        av0 = jnp.minimum(b0, av0)
        aj0 = jnp.where(m0, jnp.float32(r), aj0)
        av1 = jnp.minimum(b1, av1)
        aj1 = jnp.where(m1, jnp.float32(r + 1), aj1)

    svec = lax.broadcasted_iota(jnp.int32, (8, tm), 0).astype(jnp.float32)
    j0 = aj0 * jnp.float32(8.0) + svec
    j1 = aj1 * jnp.float32(8.0) + svec
    take1 = (av1 < av0) | ((av1 == av0) & (j1 < j0))
    av = jnp.where(take1, av1, av0)
    aj = jnp.where(take1, j1, j0)
    tmin = jnp.min(av, axis=0, keepdims=True)
    jrow = jnp.min(jnp.where(av == tmin, aj, jnp.float32(3.0e38)),
                   axis=0, keepdims=True)
    idx_ref[...] = jrow.astype(jnp.int32)[None]
    part = jnp.sum(tmin, keepdims=True)
    prev = jnp.where(i == 0, jnp.zeros_like(loss_ref[...]), loss_ref[...])
    tot = prev + part
    loss_ref[...] = jnp.where(i == ni - 1, tot * scale, tot)


def _vq_prep(w):
    k_tot, d_dim = w.shape
    nk = k_tot // _TKP
    return pl.pallas_call(
        _prep_body,
        grid=(nk,),
        in_specs=[pl.BlockSpec((_TKP, d_dim), lambda k: (k, 0))],
        out_specs=[
            pl.BlockSpec((_TKP, d_dim), lambda k: (k, 0)),
            pl.BlockSpec((_TKP, 1), lambda k: (k, 0)),
        ],
        out_shape=[
            jax.ShapeDtypeStruct((k_tot, d_dim), jnp.float32),
            jax.ShapeDtypeStruct((k_tot, 1), jnp.float32),
        ],
    )(w)


def _vq_argmin(zf, w2, wsq):
    m, d_dim = zf.shape
    k_tot = w2.shape[0]
    ni = m // _TM
    scale = (1.0 + BETA) / (m * d_dim)
    idx3, loss = pl.pallas_call(
        functools.partial(_argmin_body, ni=ni, scale=scale),
        grid=(ni,),
        in_specs=[
            pl.BlockSpec((_TM, d_dim), lambda i: (i, 0)),
            pl.BlockSpec((_TK, d_dim), lambda i: (0, 0)),
            pl.BlockSpec((_TK, 1), lambda i: (0, 0)),
        ],
        out_specs=[
            pl.BlockSpec((1, 1, _TM), lambda i: (i, 0, 0)),
            pl.BlockSpec((1, 1), lambda i: (0, 0)),
        ],
        out_shape=[
            jax.ShapeDtypeStruct((ni, 1, _TM), jnp.int32),
            jax.ShapeDtypeStruct((1, 1), jnp.float32),
        ],
        compiler_params=pltpu.CompilerParams(
            dimension_semantics=("arbitrary",)),
    )(zf, w2, wsq)
    return idx3.reshape(m), loss[0, 0]


def _gather_body(w_hbm, idx_hbm, out_hbm, idx_v, rows_a, rows_b, sem_a, sem_b):
    wid = lax.axis_index("s") * _NC + lax.axis_index("c")
    b_per_w = idx_hbm.shape[0] // _NW
    base = wid * b_per_w
    nch = b_per_w // _CHUNK

    pltpu.sync_copy(idx_hbm.at[pl.ds(base, b_per_w)], idx_v)
    bufs = (rows_a, rows_b)
    sems = (sem_a, sem_b)

    def gat(c):
        return pltpu.async_copy(
            w_hbm.at[idx_v.at[pl.ds(c * _CHUNK, _CHUNK)]],
            bufs[c % 2], sems[c % 2])

    handles = {0: gat(0), 1: gat(1)}
    for c in range(nch):
        handles[c].wait()
        pltpu.sync_copy(bufs[c % 2], out_hbm.at[pl.ds(base + c * _CHUNK, _CHUNK)])
        if c + 2 < nch:
            handles[c + 2] = gat(c + 2)


def _vq_gather(w, idx):
    m = idx.shape[0]
    d_dim = w.shape[1]
    b_per_w = m // _NW
    mesh = plsc.VectorSubcoreMesh(core_axis_name="c", subcore_axis_name="s")
    fn = functools.partial(
        pl.kernel,
        mesh=mesh,
        out_type=jax.ShapeDtypeStruct((m, d_dim), jnp.float32),
        scratch_types=[
            pltpu.VMEM((b_per_w,), jnp.int32),
            pltpu.VMEM((_CHUNK, d_dim), jnp.float32),
            pltpu.VMEM((_CHUNK, d_dim), jnp.float32),
            pltpu.SemaphoreType.DMA,
            pltpu.SemaphoreType.DMA,
        ],
    )(_gather_body)
    return fn(w, idx)


def kernel(z, W):
    zf = z.reshape(-1, z.shape[-1])
    w2, wsq = _vq_prep(W)
    idx, loss = _vq_argmin(zf, w2, wsq)
    z_q = _vq_gather(W, idx)
    z_q_st = z_q.reshape(z.shape)
    min_encoding_indices = idx.reshape(z.shape[:-1] + (1,))
    return (z_q_st, loss, min_encoding_indices)
